# baseline (speedup 1.0000x reference)
import numpy as np
import jax
import jax.numpy as jnp
from jax.experimental import pallas as pl

N = 10000
E = 160000
NODE_DIM = 128
EDGE_DIM = 16
HID = 128
N_LAYERS = 4
T = 1000


def _scheduler():
    steps = T + 1
    xs = jnp.linspace(0.0, float(T), steps)
    ac = jnp.cos((xs / T + 0.008) / (1 + 0.008) * jnp.pi * 0.5) ** 2
    ac = ac / ac[0]
    betas = jnp.clip(1.0 - ac[1:] / ac[:-1], 0.0001, 0.9999)
    acp = jnp.cumprod(1.0 - betas)
    return jnp.sqrt(acp), jnp.sqrt(1.0 - acp)


def kernel(x, pos, edge_index, edge_attr, t, Wt1, bt1, Wt2, bt2, W_in, b_in, layers):
    sqrt_acp, sqrt_1m = _scheduler()
    noise = jax.random.normal(jax.random.key(42), pos.shape, jnp.float32) * 1.0
    a = sqrt_acp[t].reshape(-1, 1, 1)
    b = sqrt_1m[t].reshape(-1, 1, 1)
    noisy_pos = a * pos + b * noise
    half = HID // 2
    freqs = jnp.exp(jnp.arange(half, dtype=jnp.float32) * -(np.log(10000.0) / (half - 1)))
    emb = t.astype(jnp.float32)[:, None] * freqs[None, :]
    emb = jnp.concatenate([jnp.sin(emb), jnp.cos(emb)], axis=-1)
    temb = jax.nn.silu(emb @ Wt1 + bt1) @ Wt2 + bt2
    reps = x.shape[0] // t.shape[0]
    h = jnp.concatenate([x, jnp.repeat(temb, reps, axis=0)], axis=-1)
    p0 = noisy_pos[0]
    p = p0
    h = h @ W_in + b_in
    src = edge_index[0]
    dst = edge_index[1]
    for lp in layers:
        rel = p[dst] - p[src]
        d2 = jnp.sum(rel * rel, axis=-1, keepdims=True)
        m = jnp.concatenate([h[dst], h[src], d2, edge_attr], axis=-1)
        m = jax.nn.silu(m @ lp['We1'] + lp['be1'])
        m = jax.nn.silu(m @ lp['We2'] + lp['be2'])
        cw = m @ lp['Wx'] + lp['bx']
        p = p + jax.ops.segment_sum(rel * cw, dst, num_segments=N) / 16.0
        agg = jax.ops.segment_sum(m, dst, num_segments=N)
        h = h + (jax.nn.silu(jnp.concatenate([h, agg], axis=-1) @ lp['Wh1'] + lp['bh1']) @ lp['Wh2'] + lp['bh2'])
    pred_noise = p - p0
    return pred_noise, noisy_pos
